# baseline (speedup 1.0000x reference)
import functools

import jax
import jax.numpy as jnp
from jax import lax
from jax.experimental import pallas as pl
from jax.experimental.pallas import tpu as pltpu
from jax.experimental.pallas import tpu_sc as plsc

N = 10000
E = 320000
K = 128
NCHUNK = E // K
NS = 16
NC = 2
CPT = 160
SB = 40
NCHUNK_PAD = CPT * NS
NPAD = 10240
ROWS_PER_TILE = NPAD // NS

RB = 1000
NG = 64
F = 128
H = 256

_HIGH = lax.Precision.HIGHEST
_DEF = lax.Precision.DEFAULT


def _sc_mesh():
    return plsc.VectorSubcoreMesh(core_axis_name="c", subcore_axis_name="s",
                                  num_cores=NC, num_subcores=NS)


def _deg_call(dst2, ones16, zrows):
    @functools.partial(
        pl.kernel,
        out_type=jax.ShapeDtypeStruct((2, NPAD, F), jnp.float32),
        mesh=_sc_mesh(),
        scratch_types=[
            pltpu.VMEM((CPT, K), jnp.int32),
            pltpu.VMEM((K, F), jnp.float32),
            pltpu.VMEM_SHARED((NPAD, F), jnp.float32),
        ],
    )
    def k(dst_hbm, ones_hbm, z_hbm, out_hbm, dstv, onesv, acc):
        c = lax.axis_index("c")
        s = lax.axis_index("s")
        pltpu.sync_copy(z_hbm, acc.at[pl.ds(s * ROWS_PER_TILE, ROWS_PER_TILE)])
        pltpu.sync_copy(ones_hbm, onesv)
        pltpu.sync_copy(dst_hbm.at[c].at[pl.ds(s * CPT, CPT)], dstv)
        plsc.subcore_barrier()

        def step(j, carry):
            pltpu.sync_copy(onesv, acc.at[dstv.at[j]], add=True)
            return carry

        lax.fori_loop(0, CPT, step, 0)
        plsc.subcore_barrier()
        sl = pl.ds(s * ROWS_PER_TILE, ROWS_PER_TILE)
        pltpu.sync_copy(acc.at[sl], out_hbm.at[c].at[sl])

    return k(dst2, ones16, zrows)


def _agg_call(g3, src2, dst2, zrows):
    @functools.partial(
        pl.kernel,
        out_type=jax.ShapeDtypeStruct((2, NPAD, F), jnp.float32),
        mesh=_sc_mesh(),
        scratch_types=[
            pltpu.VMEM((SB, K), jnp.int32),
            pltpu.VMEM((SB, K), jnp.int32),
            pltpu.VMEM((K, F), jnp.float32),
            pltpu.VMEM((K, F), jnp.float32),
            pltpu.VMEM_SHARED((NPAD, F), jnp.float32),
            pltpu.SemaphoreType.DMA,
            pltpu.SemaphoreType.DMA,
            pltpu.SemaphoreType.DMA,
            pltpu.SemaphoreType.DMA,
        ],
    )
    def k(g_hbm, src_hbm, dst_hbm, z_hbm, out_hbm, srcv, dstv, rows0, rows1,
          acc, sem0, sem1, ssem0, ssem1):
        c = lax.axis_index("c")
        s = lax.axis_index("s")
        pltpu.sync_copy(z_hbm, acc.at[pl.ds(s * ROWS_PER_TILE, ROWS_PER_TILE)])
        plsc.subcore_barrier()

        gsrc = g_hbm.at[c]

        def blk(b, carry):
            pltpu.sync_copy(src_hbm.at[pl.ds(s * CPT + b * SB, SB)], srcv)
            pltpu.sync_copy(dst_hbm.at[pl.ds(s * CPT + b * SB, SB)], dstv)

            pltpu.async_copy(gsrc.at[srcv.at[0]], rows0, sem0)
            pltpu.async_copy(gsrc.at[srcv.at[1]], rows1, sem1)

            def pair(jj, carry2):
                j0 = 2 * jj
                j1 = j0 + 1
                pltpu.make_async_copy(gsrc.at[srcv.at[j0]], rows0,
                                      sem0).wait()
                pltpu.async_copy(rows0, acc.at[dstv.at[j0]], ssem0, add=True)
                pltpu.make_async_copy(gsrc.at[srcv.at[j1]], rows1,
                                      sem1).wait()
                pltpu.async_copy(rows1, acc.at[dstv.at[j1]], ssem1, add=True)

                @pl.when(jj < SB // 2 - 1)
                def _():
                    pltpu.make_async_copy(rows0, acc.at[dstv.at[j0]],
                                          ssem0).wait()
                    pltpu.async_copy(gsrc.at[srcv.at[j0 + 2]], rows0, sem0)
                    pltpu.make_async_copy(rows1, acc.at[dstv.at[j1]],
                                          ssem1).wait()
                    pltpu.async_copy(gsrc.at[srcv.at[j1 + 2]], rows1, sem1)

                return carry2

            lax.fori_loop(0, SB // 2, pair, 0)
            pltpu.make_async_copy(rows0, acc.at[dstv.at[SB - 2]],
                                  ssem0).wait()
            pltpu.make_async_copy(rows1, acc.at[dstv.at[SB - 1]],
                                  ssem1).wait()
            return carry

        lax.fori_loop(0, CPT // SB, blk, 0)
        plsc.subcore_barrier()
        sl = pl.ds(s * ROWS_PER_TILE, ROWS_PER_TILE)
        pltpu.sync_copy(acc.at[sl], out_hbm.at[c].at[sl])

    return k(g3, src2, dst2, zrows)


def _prep_call(x, W1, b1, deg):
    def body(x_ref, w_ref, b_ref, deg_ref, g_ref, s_ref):
        h = jnp.dot(x_ref[...], w_ref[...], preferred_element_type=jnp.float32,
                    precision=_DEF)
        dis = lax.rsqrt(deg_ref[...])
        g = h * dis
        g_ref[0] = g[:, :F]
        g_ref[1] = g[:, F:]
        s_ref[...] = h * (dis * dis) + b_ref[...]

    return pl.pallas_call(
        body,
        grid=(N // RB,),
        in_specs=[
            pl.BlockSpec((RB, F), lambda i: (i, 0)),
            pl.BlockSpec((F, H), lambda i: (0, 0)),
            pl.BlockSpec((1, H), lambda i: (0, 0)),
            pl.BlockSpec((RB, 1), lambda i: (i, 0)),
        ],
        out_specs=[
            pl.BlockSpec((2, RB, F), lambda i: (0, i, 0)),
            pl.BlockSpec((RB, H), lambda i: (i, 0)),
        ],
        out_shape=[
            jax.ShapeDtypeStruct((2, N, F), jnp.float32),
            jax.ShapeDtypeStruct((N, H), jnp.float32),
        ],
    )(x, W1, b1, deg)


def _mid_call(P, sarr, deg, W2, b2):
    def body(p_ref, s_ref, deg_ref, w_ref, b_ref, g_ref, s2_ref):
        pfull = jnp.concatenate([p_ref[0], p_ref[1]], axis=1)
        dis = lax.rsqrt(deg_ref[...])
        out1 = jnp.maximum(pfull * dis + s_ref[...], 0.0)
        h2 = jnp.dot(out1, w_ref[...], preferred_element_type=jnp.float32,
                     precision=_DEF)
        g2 = h2 * dis
        g_ref[0] = g2[:, :F]
        g_ref[1] = g2[:, F:]
        s2_ref[...] = h2 * (dis * dis) + b_ref[...]

    return pl.pallas_call(
        body,
        grid=(N // RB,),
        in_specs=[
            pl.BlockSpec((2, RB, F), lambda i: (0, i, 0)),
            pl.BlockSpec((RB, H), lambda i: (i, 0)),
            pl.BlockSpec((RB, 1), lambda i: (i, 0)),
            pl.BlockSpec((H, H), lambda i: (0, 0)),
            pl.BlockSpec((1, H), lambda i: (0, 0)),
        ],
        out_specs=[
            pl.BlockSpec((2, RB, F), lambda i: (0, i, 0)),
            pl.BlockSpec((RB, H), lambda i: (i, 0)),
        ],
        out_shape=[
            jax.ShapeDtypeStruct((2, N, F), jnp.float32),
            jax.ShapeDtypeStruct((N, H), jnp.float32),
        ],
    )(P, sarr, deg, W2, b2)


def _pool_call(P, sarr, deg, batch, W3, b3):
    nsteps = N // RB

    def body(p_ref, s_ref, deg_ref, b_ref, w3_ref, b3_ref, emb_ref,
             acc_ref, cnt_ref):
        i = pl.program_id(0)

        @pl.when(i == 0)
        def _():
            acc_ref[...] = jnp.zeros_like(acc_ref)
            cnt_ref[...] = jnp.zeros_like(cnt_ref)

        pfull = jnp.concatenate([p_ref[0], p_ref[1]], axis=1)
        dis = lax.rsqrt(deg_ref[...])
        out2 = jnp.maximum(pfull * dis + s_ref[...], 0.0)
        oh = (b_ref[...] == lax.broadcasted_iota(jnp.int32, (RB, NG), 1)
              ).astype(jnp.float32)
        acc_ref[...] += lax.dot_general(oh, out2, (((0,), (0,)), ((), ())),
                                        precision=_HIGH)
        cnt_ref[...] += lax.dot_general(oh, jnp.ones((RB, 1), jnp.float32),
                                        (((0,), (0,)), ((), ())),
                                        precision=_HIGH)

        @pl.when(i == nsteps - 1)
        def _():
            pooled = acc_ref[...] / jnp.maximum(cnt_ref[...], 1.0)
            emb_ref[...] = jnp.dot(pooled, w3_ref[...],
                                   preferred_element_type=jnp.float32,
                                   precision=_DEF) + b3_ref[...]

    return pl.pallas_call(
        body,
        grid=(nsteps,),
        in_specs=[
            pl.BlockSpec((2, RB, F), lambda i: (0, i, 0)),
            pl.BlockSpec((RB, H), lambda i: (i, 0)),
            pl.BlockSpec((RB, 1), lambda i: (i, 0)),
            pl.BlockSpec((RB, 1), lambda i: (i, 0)),
            pl.BlockSpec((H, F), lambda i: (0, 0)),
            pl.BlockSpec((1, F), lambda i: (0, 0)),
        ],
        out_specs=pl.BlockSpec((NG, F), lambda i: (0, 0)),
        out_shape=jax.ShapeDtypeStruct((NG, F), jnp.float32),
        scratch_shapes=[
            pltpu.VMEM((NG, H), jnp.float32),
            pltpu.VMEM((NG, 1), jnp.float32),
        ],
    )(P, sarr, deg, batch, W3, b3)


def _head_call(s_emb, q_emb, labels):
    def body(se_ref, qe_ref, lab_ref, lp_ref, pred_ref):
        m = (lab_ref[...] == lax.broadcasted_iota(jnp.int32, (NG, 2), 1)
             ).astype(jnp.float32)
        sums = lax.dot_general(m, se_ref[...], (((0,), (0,)), ((), ())),
                               precision=_HIGH)
        cnts = lax.dot_general(m, jnp.ones((NG, 1), jnp.float32),
                               (((0,), (0,)), ((), ())), precision=_HIGH)
        protos = sums / jnp.maximum(cnts, 1.0)
        q = qe_ref[...]
        d0 = q - protos[0:1, :]
        d1 = q - protos[1:2, :]
        d20 = jnp.sum(d0 * d0, axis=1, keepdims=True)
        d21 = jnp.sum(d1 * d1, axis=1, keepdims=True)
        d2 = jnp.concatenate([d20, d21], axis=1)
        dists = jnp.sqrt(jnp.maximum(d2, 1e-12))
        neg = -dists
        mx = jnp.max(neg, axis=1, keepdims=True)
        lse = mx + jnp.log(jnp.sum(jnp.exp(neg - mx), axis=1, keepdims=True))
        lp_ref[...] = neg - lse
        pred_ref[...] = (dists[:, 1:2] < dists[:, 0:1]).astype(jnp.int32)

    return pl.pallas_call(
        body,
        out_shape=[
            jax.ShapeDtypeStruct((NG, 2), jnp.float32),
            jax.ShapeDtypeStruct((NG, 1), jnp.int32),
        ],
    )(s_emb, q_emb, labels)


def _encode(x, src2, dst2, deg, batch, W1, b1, W2, b2, W3, b3, z128):
    g1, s1 = _prep_call(x, W1, b1, deg)
    P1 = _agg_call(g1, src2, dst2, z128)[:, :N, :]
    g2, s2 = _mid_call(P1, s1, deg, W2, b2)
    P2 = _agg_call(g2, src2, dst2, z128)[:, :N, :]
    return _pool_call(P2, s2, deg, batch, W3, b3)


def _pad_chunks(a, garbage):
    a = a.reshape(NCHUNK, K)
    npadrows = NCHUNK_PAD - NCHUNK
    if garbage:
        fill = N + (jnp.arange(npadrows * K, dtype=jnp.int32) % (NPAD - N))
    else:
        fill = jnp.arange(npadrows * K, dtype=jnp.int32) % N
    return jnp.concatenate([a, fill.reshape(npadrows, K)], axis=0)


def kernel(support_x, support_edge_index, support_batch, support_labels,
           query_x, query_edge_index, query_batch, W1, b1, W2, b2, W3, b3):
    s_src = _pad_chunks(support_edge_index[0], False)
    s_dst = _pad_chunks(support_edge_index[1], True)
    q_src = _pad_chunks(query_edge_index[0], False)
    q_dst = _pad_chunks(query_edge_index[1], True)

    ones_rows = jnp.ones((K, F), jnp.float32)
    z128 = jnp.zeros((ROWS_PER_TILE, F), jnp.float32)

    deg2 = _deg_call(jnp.stack([s_dst, q_dst]), ones_rows, z128)
    s_deg = deg2[0, :N, 0:1] + 1.0
    q_deg = deg2[1, :N, 0:1] + 1.0

    b1r = b1.reshape(1, H)
    b2r = b2.reshape(1, H)
    b3r = b3.reshape(1, F)

    s_emb = _encode(support_x, s_src, s_dst, s_deg,
                    support_batch.reshape(N, 1), W1, b1r, W2, b2r, W3, b3r,
                    z128)
    q_emb = _encode(query_x, q_src, q_dst, q_deg,
                    query_batch.reshape(N, 1), W1, b1r, W2, b2r, W3, b3r,
                    z128)

    log_probs, pred = _head_call(s_emb, q_emb, support_labels.reshape(NG, 1))
    return (log_probs, pred.reshape(NG))


# speedup vs baseline: 1.2514x; 1.2514x over previous
import functools

import jax
import jax.numpy as jnp
from jax import lax
from jax.experimental import pallas as pl
from jax.experimental.pallas import tpu as pltpu
from jax.experimental.pallas import tpu_sc as plsc

N = 10000
E = 320000
K = 128
NCHUNK = E // K
NS = 16
NC = 2
CPT = 160
SB = 40
NCHUNK_PAD = CPT * NS
NPAD = 10240
ROWS_PER_TILE = NPAD // NS

RB = 1000
NG = 64
F = 128
H = 256

_HIGH = lax.Precision.HIGHEST
_DEF = lax.Precision.DEFAULT


def _sc_mesh():
    return plsc.VectorSubcoreMesh(core_axis_name="c", subcore_axis_name="s",
                                  num_cores=NC, num_subcores=NS)


def _deg_call(dst2, ones16, zrows):
    @functools.partial(
        pl.kernel,
        out_type=jax.ShapeDtypeStruct((2, NPAD, F), jnp.float32),
        mesh=_sc_mesh(),
        scratch_types=[
            pltpu.VMEM((CPT, K), jnp.int32),
            pltpu.VMEM((K, F), jnp.float32),
            pltpu.VMEM_SHARED((NPAD, F), jnp.float32),
        ],
    )
    def k(dst_hbm, ones_hbm, z_hbm, out_hbm, dstv, onesv, acc):
        c = lax.axis_index("c")
        s = lax.axis_index("s")
        pltpu.sync_copy(z_hbm, acc.at[pl.ds(s * ROWS_PER_TILE, ROWS_PER_TILE)])
        pltpu.sync_copy(ones_hbm, onesv)
        pltpu.sync_copy(dst_hbm.at[c].at[pl.ds(s * CPT, CPT)], dstv)
        plsc.subcore_barrier()

        def step(j, carry):
            pltpu.sync_copy(onesv, acc.at[dstv.at[j]], add=True)
            return carry

        lax.fori_loop(0, CPT, step, 0)
        plsc.subcore_barrier()
        sl = pl.ds(s * ROWS_PER_TILE, ROWS_PER_TILE)
        pltpu.sync_copy(acc.at[sl], out_hbm.at[c].at[sl])

    return k(dst2, ones16, zrows)


def _agg_call(g3, src2, dst2, zrows):
    @functools.partial(
        pl.kernel,
        out_type=jax.ShapeDtypeStruct((2, NPAD, F), jnp.float32),
        mesh=_sc_mesh(),
        scratch_types=[
            pltpu.VMEM((SB, K), jnp.int32),
            pltpu.VMEM((SB, K), jnp.int32),
            pltpu.VMEM((K, F), jnp.float32),
            pltpu.VMEM((K, F), jnp.float32),
            pltpu.VMEM_SHARED((NPAD, F), jnp.float32),
            pltpu.SemaphoreType.DMA,
            pltpu.SemaphoreType.DMA,
        ],
    )
    def k(g_hbm, src_hbm, dst_hbm, z_hbm, out_hbm, srcv, dstv, rows0, rows1,
          acc, sem0, sem1):
        c = lax.axis_index("c")
        s = lax.axis_index("s")
        pltpu.sync_copy(z_hbm, acc.at[pl.ds(s * ROWS_PER_TILE, ROWS_PER_TILE)])
        plsc.subcore_barrier()

        gsrc = g_hbm.at[c]

        def blk(b, carry):
            pltpu.sync_copy(src_hbm.at[pl.ds(s * CPT + b * SB, SB)], srcv)
            pltpu.sync_copy(dst_hbm.at[pl.ds(s * CPT + b * SB, SB)], dstv)

            bcnt = jnp.int32(SB)

            @pl.when(bcnt > 0)
            def _():
                pltpu.async_copy(gsrc.at[srcv.at[0]], rows0, sem0)

            def pair(jj, carry2):
                j0 = 2 * jj
                j1 = j0 + 1

                @pl.when(j1 < bcnt)
                def _():
                    pltpu.async_copy(gsrc.at[srcv.at[j1]], rows1, sem1)

                @pl.when(j0 < bcnt)
                def _():
                    pltpu.make_async_copy(gsrc.at[srcv.at[j0]], rows0,
                                          sem0).wait()
                    pltpu.sync_copy(rows0, acc.at[dstv.at[j0]], add=True)

                @pl.when(j0 + 2 < bcnt)
                def _():
                    pltpu.async_copy(gsrc.at[srcv.at[j0 + 2]], rows0, sem0)

                @pl.when(j1 < bcnt)
                def _():
                    pltpu.make_async_copy(gsrc.at[srcv.at[j1]], rows1,
                                          sem1).wait()
                    pltpu.sync_copy(rows1, acc.at[dstv.at[j1]], add=True)

                return carry2

            lax.fori_loop(0, SB // 2, pair, 0)
            return carry

        lax.fori_loop(0, CPT // SB, blk, 0)
        plsc.subcore_barrier()
        sl = pl.ds(s * ROWS_PER_TILE, ROWS_PER_TILE)
        pltpu.sync_copy(acc.at[sl], out_hbm.at[c].at[sl])

    return k(g3, src2, dst2, zrows)


def _prep_call(x, W1, b1, deg):
    def body(x_ref, w_ref, b_ref, deg_ref, g_ref, s_ref):
        h = jnp.dot(x_ref[...], w_ref[...], preferred_element_type=jnp.float32,
                    precision=_DEF)
        dis = lax.rsqrt(deg_ref[...])
        g = h * dis
        g_ref[0] = g[:, :F]
        g_ref[1] = g[:, F:]
        s_ref[...] = h * (dis * dis) + b_ref[...]

    return pl.pallas_call(
        body,
        grid=(N // RB,),
        in_specs=[
            pl.BlockSpec((RB, F), lambda i: (i, 0)),
            pl.BlockSpec((F, H), lambda i: (0, 0)),
            pl.BlockSpec((1, H), lambda i: (0, 0)),
            pl.BlockSpec((RB, 1), lambda i: (i, 0)),
        ],
        out_specs=[
            pl.BlockSpec((2, RB, F), lambda i: (0, i, 0)),
            pl.BlockSpec((RB, H), lambda i: (i, 0)),
        ],
        out_shape=[
            jax.ShapeDtypeStruct((2, N, F), jnp.float32),
            jax.ShapeDtypeStruct((N, H), jnp.float32),
        ],
    )(x, W1, b1, deg)


def _mid_call(P, sarr, deg, W2, b2):
    def body(p_ref, s_ref, deg_ref, w_ref, b_ref, g_ref, s2_ref):
        pfull = jnp.concatenate([p_ref[0], p_ref[1]], axis=1)
        dis = lax.rsqrt(deg_ref[...])
        out1 = jnp.maximum(pfull * dis + s_ref[...], 0.0)
        h2 = jnp.dot(out1, w_ref[...], preferred_element_type=jnp.float32,
                     precision=_DEF)
        g2 = h2 * dis
        g_ref[0] = g2[:, :F]
        g_ref[1] = g2[:, F:]
        s2_ref[...] = h2 * (dis * dis) + b_ref[...]

    return pl.pallas_call(
        body,
        grid=(N // RB,),
        in_specs=[
            pl.BlockSpec((2, RB, F), lambda i: (0, i, 0)),
            pl.BlockSpec((RB, H), lambda i: (i, 0)),
            pl.BlockSpec((RB, 1), lambda i: (i, 0)),
            pl.BlockSpec((H, H), lambda i: (0, 0)),
            pl.BlockSpec((1, H), lambda i: (0, 0)),
        ],
        out_specs=[
            pl.BlockSpec((2, RB, F), lambda i: (0, i, 0)),
            pl.BlockSpec((RB, H), lambda i: (i, 0)),
        ],
        out_shape=[
            jax.ShapeDtypeStruct((2, N, F), jnp.float32),
            jax.ShapeDtypeStruct((N, H), jnp.float32),
        ],
    )(P, sarr, deg, W2, b2)


def _pool_call(P, sarr, deg, batch, W3, b3):
    nsteps = N // RB

    def body(p_ref, s_ref, deg_ref, b_ref, w3_ref, b3_ref, emb_ref,
             acc_ref, cnt_ref):
        i = pl.program_id(0)

        @pl.when(i == 0)
        def _():
            acc_ref[...] = jnp.zeros_like(acc_ref)
            cnt_ref[...] = jnp.zeros_like(cnt_ref)

        pfull = jnp.concatenate([p_ref[0], p_ref[1]], axis=1)
        dis = lax.rsqrt(deg_ref[...])
        out2 = jnp.maximum(pfull * dis + s_ref[...], 0.0)
        oh = (b_ref[...] == lax.broadcasted_iota(jnp.int32, (RB, NG), 1)
              ).astype(jnp.float32)
        acc_ref[...] += lax.dot_general(oh, out2, (((0,), (0,)), ((), ())),
                                        precision=_HIGH)
        cnt_ref[...] += lax.dot_general(oh, jnp.ones((RB, 1), jnp.float32),
                                        (((0,), (0,)), ((), ())),
                                        precision=_HIGH)

        @pl.when(i == nsteps - 1)
        def _():
            pooled = acc_ref[...] / jnp.maximum(cnt_ref[...], 1.0)
            emb_ref[...] = jnp.dot(pooled, w3_ref[...],
                                   preferred_element_type=jnp.float32,
                                   precision=_DEF) + b3_ref[...]

    return pl.pallas_call(
        body,
        grid=(nsteps,),
        in_specs=[
            pl.BlockSpec((2, RB, F), lambda i: (0, i, 0)),
            pl.BlockSpec((RB, H), lambda i: (i, 0)),
            pl.BlockSpec((RB, 1), lambda i: (i, 0)),
            pl.BlockSpec((RB, 1), lambda i: (i, 0)),
            pl.BlockSpec((H, F), lambda i: (0, 0)),
            pl.BlockSpec((1, F), lambda i: (0, 0)),
        ],
        out_specs=pl.BlockSpec((NG, F), lambda i: (0, 0)),
        out_shape=jax.ShapeDtypeStruct((NG, F), jnp.float32),
        scratch_shapes=[
            pltpu.VMEM((NG, H), jnp.float32),
            pltpu.VMEM((NG, 1), jnp.float32),
        ],
    )(P, sarr, deg, batch, W3, b3)


def _head_call(s_emb, q_emb, labels):
    def body(se_ref, qe_ref, lab_ref, lp_ref, pred_ref):
        m = (lab_ref[...] == lax.broadcasted_iota(jnp.int32, (NG, 2), 1)
             ).astype(jnp.float32)
        sums = lax.dot_general(m, se_ref[...], (((0,), (0,)), ((), ())),
                               precision=_HIGH)
        cnts = lax.dot_general(m, jnp.ones((NG, 1), jnp.float32),
                               (((0,), (0,)), ((), ())), precision=_HIGH)
        protos = sums / jnp.maximum(cnts, 1.0)
        q = qe_ref[...]
        d0 = q - protos[0:1, :]
        d1 = q - protos[1:2, :]
        d20 = jnp.sum(d0 * d0, axis=1, keepdims=True)
        d21 = jnp.sum(d1 * d1, axis=1, keepdims=True)
        d2 = jnp.concatenate([d20, d21], axis=1)
        dists = jnp.sqrt(jnp.maximum(d2, 1e-12))
        neg = -dists
        mx = jnp.max(neg, axis=1, keepdims=True)
        lse = mx + jnp.log(jnp.sum(jnp.exp(neg - mx), axis=1, keepdims=True))
        lp_ref[...] = neg - lse
        pred_ref[...] = (dists[:, 1:2] < dists[:, 0:1]).astype(jnp.int32)

    return pl.pallas_call(
        body,
        out_shape=[
            jax.ShapeDtypeStruct((NG, 2), jnp.float32),
            jax.ShapeDtypeStruct((NG, 1), jnp.int32),
        ],
    )(s_emb, q_emb, labels)


def _encode(x, src2, dst2, deg, batch, W1, b1, W2, b2, W3, b3, z128):
    g1, s1 = _prep_call(x, W1, b1, deg)
    P1 = _agg_call(g1, src2, dst2, z128)[:, :N, :]
    g2, s2 = _mid_call(P1, s1, deg, W2, b2)
    P2 = _agg_call(g2, src2, dst2, z128)[:, :N, :]
    return _pool_call(P2, s2, deg, batch, W3, b3)


def _pad_chunks(a, garbage):
    a = a.reshape(NCHUNK, K)
    npadrows = NCHUNK_PAD - NCHUNK
    if garbage:
        fill = N + (jnp.arange(npadrows * K, dtype=jnp.int32) % (NPAD - N))
    else:
        fill = jnp.arange(npadrows * K, dtype=jnp.int32) % N
    return jnp.concatenate([a, fill.reshape(npadrows, K)], axis=0)


def kernel(support_x, support_edge_index, support_batch, support_labels,
           query_x, query_edge_index, query_batch, W1, b1, W2, b2, W3, b3):
    s_src = _pad_chunks(support_edge_index[0], False)
    s_dst = _pad_chunks(support_edge_index[1], True)
    q_src = _pad_chunks(query_edge_index[0], False)
    q_dst = _pad_chunks(query_edge_index[1], True)

    ones_rows = jnp.ones((K, F), jnp.float32)
    z128 = jnp.zeros((ROWS_PER_TILE, F), jnp.float32)

    deg2 = _deg_call(jnp.stack([s_dst, q_dst]), ones_rows, z128)
    s_deg = deg2[0, :N, 0:1] + 1.0
    q_deg = deg2[1, :N, 0:1] + 1.0

    b1r = b1.reshape(1, H)
    b2r = b2.reshape(1, H)
    b3r = b3.reshape(1, F)

    s_emb = _encode(support_x, s_src, s_dst, s_deg,
                    support_batch.reshape(N, 1), W1, b1r, W2, b2r, W3, b3r,
                    z128)
    q_emb = _encode(query_x, q_src, q_dst, q_deg,
                    query_batch.reshape(N, 1), W1, b1r, W2, b2r, W3, b3r,
                    z128)

    log_probs, pred = _head_call(s_emb, q_emb, support_labels.reshape(NG, 1))
    return (log_probs, pred.reshape(NG))


# speedup vs baseline: 1.2839x; 1.0259x over previous
import functools

import jax
import jax.numpy as jnp
from jax import lax
from jax.experimental import pallas as pl
from jax.experimental.pallas import tpu as pltpu
from jax.experimental.pallas import tpu_sc as plsc

N = 10000
E = 320000
K = 128
NCHUNK = E // K
NS = 16
NC = 2
CPT = 160
SB = 40
NCHUNK_PAD = CPT * NS
NPAD = 10240
ROWS_PER_TILE = NPAD // NS

RB = 1000
NG = 64
F = 128
H = 256

_HIGH = lax.Precision.HIGHEST
_DEF = lax.Precision.DEFAULT


def _sc_mesh():
    return plsc.VectorSubcoreMesh(core_axis_name="c", subcore_axis_name="s",
                                  num_cores=NC, num_subcores=NS)


def _deg_call(dst2, ones16, zrows):
    @functools.partial(
        pl.kernel,
        out_type=jax.ShapeDtypeStruct((2, NPAD, F), jnp.float32),
        mesh=_sc_mesh(),
        scratch_types=[
            pltpu.VMEM((CPT, K), jnp.int32),
            pltpu.VMEM((K, F), jnp.float32),
            pltpu.VMEM_SHARED((NPAD, F), jnp.float32),
        ],
    )
    def k(dst_hbm, ones_hbm, z_hbm, out_hbm, dstv, onesv, acc):
        c = lax.axis_index("c")
        s = lax.axis_index("s")
        pltpu.sync_copy(z_hbm, acc.at[pl.ds(s * ROWS_PER_TILE, ROWS_PER_TILE)])
        pltpu.sync_copy(ones_hbm, onesv)
        pltpu.sync_copy(dst_hbm.at[c].at[pl.ds(s * CPT, CPT)], dstv)
        plsc.subcore_barrier()

        def step(j, carry):
            pltpu.sync_copy(onesv, acc.at[dstv.at[j]], add=True)
            return carry

        lax.fori_loop(0, CPT, step, 0)
        plsc.subcore_barrier()
        sl = pl.ds(s * ROWS_PER_TILE, ROWS_PER_TILE)
        pltpu.sync_copy(acc.at[sl], out_hbm.at[c].at[sl])

    return k(dst2, ones16, zrows)


def _agg_call(g3, src2, dst2, zrows):
    @functools.partial(
        pl.kernel,
        out_type=jax.ShapeDtypeStruct((2, NPAD, F), jnp.float32),
        mesh=_sc_mesh(),
        scratch_types=[
            pltpu.VMEM((SB, K), jnp.int32),
            pltpu.VMEM((SB, K), jnp.int32),
            pltpu.VMEM((K, F), jnp.float32),
            pltpu.VMEM((K, F), jnp.float32),
            pltpu.VMEM_SHARED((NPAD, F), jnp.float32),
            pltpu.SemaphoreType.DMA,
            pltpu.SemaphoreType.DMA,
        ],
    )
    def k(g_hbm, src_hbm, dst_hbm, z_hbm, out_hbm, srcv, dstv, rows0, rows1,
          acc, sem0, sem1):
        c = lax.axis_index("c")
        s = lax.axis_index("s")
        pltpu.sync_copy(z_hbm, acc.at[pl.ds(s * ROWS_PER_TILE, ROWS_PER_TILE)])
        plsc.subcore_barrier()

        gsrc = g_hbm.at[c]

        def blk(b, carry):
            pltpu.sync_copy(src_hbm.at[pl.ds(s * CPT + b * SB, SB)], srcv)
            pltpu.sync_copy(dst_hbm.at[pl.ds(s * CPT + b * SB, SB)], dstv)

            bcnt = jnp.int32(SB)

            @pl.when(bcnt > 0)
            def _():
                pltpu.async_copy(gsrc.at[srcv.at[0]], rows0, sem0)

            def pair(jj, carry2):
                j0 = 2 * jj
                j1 = j0 + 1

                @pl.when(j1 < bcnt)
                def _():
                    pltpu.async_copy(gsrc.at[srcv.at[j1]], rows1, sem1)

                @pl.when(j0 < bcnt)
                def _():
                    pltpu.make_async_copy(gsrc.at[srcv.at[j0]], rows0,
                                          sem0).wait()
                    pltpu.sync_copy(rows0, acc.at[dstv.at[j0]], add=True)

                @pl.when(j0 + 2 < bcnt)
                def _():
                    pltpu.async_copy(gsrc.at[srcv.at[j0 + 2]], rows0, sem0)

                @pl.when(j1 < bcnt)
                def _():
                    pltpu.make_async_copy(gsrc.at[srcv.at[j1]], rows1,
                                          sem1).wait()
                    pltpu.sync_copy(rows1, acc.at[dstv.at[j1]], add=True)

                return carry2

            lax.fori_loop(0, SB // 2, pair, 0)
            return carry

        lax.fori_loop(0, CPT // SB, blk, 0)
        plsc.subcore_barrier()
        sl = pl.ds(s * ROWS_PER_TILE, ROWS_PER_TILE)
        pltpu.sync_copy(acc.at[sl], out_hbm.at[c].at[sl])

    return k(g3, src2, dst2, zrows)


def _prep_call(x, W1, b1, deg):
    def body(x_ref, w_ref, b_ref, deg_ref, g_ref, s_ref):
        h = jnp.dot(x_ref[...], w_ref[...], preferred_element_type=jnp.float32,
                    precision=_DEF)
        dis = lax.rsqrt(deg_ref[...])
        g = h * dis
        g_ref[0] = g[:, :F]
        g_ref[1] = g[:, F:]
        s_ref[...] = h * (dis * dis) + b_ref[...]

    return pl.pallas_call(
        body,
        grid=(N // RB,),
        in_specs=[
            pl.BlockSpec((RB, F), lambda i: (i, 0)),
            pl.BlockSpec((F, H), lambda i: (0, 0)),
            pl.BlockSpec((1, H), lambda i: (0, 0)),
            pl.BlockSpec((RB, 1), lambda i: (i, 0)),
        ],
        out_specs=[
            pl.BlockSpec((2, RB, F), lambda i: (0, i, 0)),
            pl.BlockSpec((RB, H), lambda i: (i, 0)),
        ],
        out_shape=[
            jax.ShapeDtypeStruct((2, N, F), jnp.float32),
            jax.ShapeDtypeStruct((N, H), jnp.float32),
        ],
    )(x, W1, b1, deg)


def _mid_call(P, sarr, deg, W2, b2):
    def body(p_ref, s_ref, deg_ref, w_ref, b_ref, g_ref, s2_ref):
        pfull = jnp.concatenate([p_ref[0], p_ref[1]], axis=1)
        dis = lax.rsqrt(deg_ref[...])
        out1 = jnp.maximum(pfull * dis + s_ref[...], 0.0)
        h2 = jnp.dot(out1, w_ref[...], preferred_element_type=jnp.float32,
                     precision=_DEF)
        g2 = h2 * dis
        g_ref[0] = g2[:, :F]
        g_ref[1] = g2[:, F:]
        s2_ref[...] = h2 * (dis * dis) + b_ref[...]

    return pl.pallas_call(
        body,
        grid=(N // RB,),
        in_specs=[
            pl.BlockSpec((2, RB, F), lambda i: (0, i, 0)),
            pl.BlockSpec((RB, H), lambda i: (i, 0)),
            pl.BlockSpec((RB, 1), lambda i: (i, 0)),
            pl.BlockSpec((H, H), lambda i: (0, 0)),
            pl.BlockSpec((1, H), lambda i: (0, 0)),
        ],
        out_specs=[
            pl.BlockSpec((2, RB, F), lambda i: (0, i, 0)),
            pl.BlockSpec((RB, H), lambda i: (i, 0)),
        ],
        out_shape=[
            jax.ShapeDtypeStruct((2, N, F), jnp.float32),
            jax.ShapeDtypeStruct((N, H), jnp.float32),
        ],
    )(P, sarr, deg, W2, b2)


def _pool_call(P, sarr, deg, batch, W3, b3):
    nsteps = N // RB

    def body(p_ref, s_ref, deg_ref, b_ref, w3_ref, b3_ref, emb_ref,
             acc_ref, cnt_ref):
        i = pl.program_id(0)

        @pl.when(i == 0)
        def _():
            acc_ref[...] = jnp.zeros_like(acc_ref)
            cnt_ref[...] = jnp.zeros_like(cnt_ref)

        pfull = jnp.concatenate([p_ref[0], p_ref[1]], axis=1)
        dis = lax.rsqrt(deg_ref[...])
        out2 = jnp.maximum(pfull * dis + s_ref[...], 0.0)
        oh = (b_ref[...] == lax.broadcasted_iota(jnp.int32, (RB, NG), 1)
              ).astype(jnp.float32)
        acc_ref[...] += lax.dot_general(oh, out2, (((0,), (0,)), ((), ())),
                                        precision=_HIGH)
        cnt_ref[...] += lax.dot_general(oh, jnp.ones((RB, 1), jnp.float32),
                                        (((0,), (0,)), ((), ())),
                                        precision=_HIGH)

        @pl.when(i == nsteps - 1)
        def _():
            pooled = acc_ref[...] / jnp.maximum(cnt_ref[...], 1.0)
            emb_ref[...] = jnp.dot(pooled, w3_ref[...],
                                   preferred_element_type=jnp.float32,
                                   precision=_DEF) + b3_ref[...]

    return pl.pallas_call(
        body,
        grid=(nsteps,),
        in_specs=[
            pl.BlockSpec((2, RB, F), lambda i: (0, i, 0)),
            pl.BlockSpec((RB, H), lambda i: (i, 0)),
            pl.BlockSpec((RB, 1), lambda i: (i, 0)),
            pl.BlockSpec((RB, 1), lambda i: (i, 0)),
            pl.BlockSpec((H, F), lambda i: (0, 0)),
            pl.BlockSpec((1, F), lambda i: (0, 0)),
        ],
        out_specs=pl.BlockSpec((NG, F), lambda i: (0, 0)),
        out_shape=jax.ShapeDtypeStruct((NG, F), jnp.float32),
        scratch_shapes=[
            pltpu.VMEM((NG, H), jnp.float32),
            pltpu.VMEM((NG, 1), jnp.float32),
        ],
    )(P, sarr, deg, batch, W3, b3)


def _head_call(s_emb, q_emb, labels):
    def body(se_ref, qe_ref, lab_ref, lp_ref, pred_ref):
        m = (lab_ref[...] == lax.broadcasted_iota(jnp.int32, (NG, 2), 1)
             ).astype(jnp.float32)
        sums = lax.dot_general(m, se_ref[...], (((0,), (0,)), ((), ())),
                               precision=_HIGH)
        cnts = lax.dot_general(m, jnp.ones((NG, 1), jnp.float32),
                               (((0,), (0,)), ((), ())), precision=_HIGH)
        protos = sums / jnp.maximum(cnts, 1.0)
        q = qe_ref[...]
        d0 = q - protos[0:1, :]
        d1 = q - protos[1:2, :]
        d20 = jnp.sum(d0 * d0, axis=1, keepdims=True)
        d21 = jnp.sum(d1 * d1, axis=1, keepdims=True)
        d2 = jnp.concatenate([d20, d21], axis=1)
        dists = jnp.sqrt(jnp.maximum(d2, 1e-12))
        neg = -dists
        mx = jnp.max(neg, axis=1, keepdims=True)
        lse = mx + jnp.log(jnp.sum(jnp.exp(neg - mx), axis=1, keepdims=True))
        lp_ref[...] = neg - lse
        pred_ref[...] = (dists[:, 1:2] < dists[:, 0:1]).astype(jnp.int32)

    return pl.pallas_call(
        body,
        out_shape=[
            jax.ShapeDtypeStruct((NG, 2), jnp.float32),
            jax.ShapeDtypeStruct((NG, 1), jnp.int32),
        ],
    )(s_emb, q_emb, labels)


def _encode(x, src2, dst2, deg, batch, W1, b1, W2, b2, W3, b3, z128):
    g1, s1 = _prep_call(x, W1, b1, deg)
    P1 = _agg_call(g1, src2, dst2, z128)
    g2, s2 = _mid_call(P1, s1, deg, W2, b2)
    P2 = _agg_call(g2, src2, dst2, z128)
    return _pool_call(P2, s2, deg, batch, W3, b3)


def _pad_chunks(a, garbage):
    a = a.reshape(NCHUNK, K)
    npadrows = NCHUNK_PAD - NCHUNK
    if garbage:
        fill = N + (jnp.arange(npadrows * K, dtype=jnp.int32) % (NPAD - N))
    else:
        fill = jnp.arange(npadrows * K, dtype=jnp.int32) % N
    return jnp.concatenate([a, fill.reshape(npadrows, K)], axis=0)


def kernel(support_x, support_edge_index, support_batch, support_labels,
           query_x, query_edge_index, query_batch, W1, b1, W2, b2, W3, b3):
    s_src = _pad_chunks(support_edge_index[0], False)
    s_dst = _pad_chunks(support_edge_index[1], True)
    q_src = _pad_chunks(query_edge_index[0], False)
    q_dst = _pad_chunks(query_edge_index[1], True)

    ones_rows = jnp.ones((K, F), jnp.float32)
    z128 = jnp.zeros((ROWS_PER_TILE, F), jnp.float32)

    deg2 = _deg_call(jnp.stack([s_dst, q_dst]), ones_rows, z128)
    s_deg = deg2[0, :N, 0:1] + 1.0
    q_deg = deg2[1, :N, 0:1] + 1.0

    b1r = b1.reshape(1, H)
    b2r = b2.reshape(1, H)
    b3r = b3.reshape(1, F)

    s_emb = _encode(support_x, s_src, s_dst, s_deg,
                    support_batch.reshape(N, 1), W1, b1r, W2, b2r, W3, b3r,
                    z128)
    q_emb = _encode(query_x, q_src, q_dst, q_deg,
                    query_batch.reshape(N, 1), W1, b1r, W2, b2r, W3, b3r,
                    z128)

    log_probs, pred = _head_call(s_emb, q_emb, support_labels.reshape(NG, 1))
    return (log_probs, pred.reshape(NG))
